# Initial kernel scaffold; baseline (speedup 1.0000x reference)
import functools

import jax
import jax.numpy as jnp
from jax import lax
from jax.experimental import pallas as pl
from jax.experimental.pallas import tpu as pltpu
from jax.experimental.pallas import tpu_sc as plsc

BATCH = 1024
SEQ = 200
VOCAB = 100000
NUM_CLASSES = 2
NUM_SUBCORES = 16
ROWS_PER_WORKER = BATCH // NUM_SUBCORES
TOKENS_PER_WORKER = ROWS_PER_WORKER * SEQ
ROW_GROUPS = ROWS_PER_WORKER // 16


def _sc_body(ids_hbm, w_hbm, b_hbm, out_hbm, ids_v, w_v, b_v, out_v):
    c = lax.axis_index("c")
    s = lax.axis_index("s")

    pltpu.sync_copy(w_hbm.at[pl.ds(c * VOCAB, VOCAB)], w_v)
    pltpu.sync_copy(ids_hbm.at[pl.ds(s * TOKENS_PER_WORKER, TOKENS_PER_WORKER)],
                    ids_v)
    pltpu.sync_copy(b_hbm.at[pl.ds(c * 16, 16)], b_v)

    bvec = b_v[...]
    row_off = lax.iota(jnp.int32, 16) * SEQ

    for g in range(ROW_GROUPS):
        base = row_off + g * (16 * SEQ)

        def body_l(l, acc, base=base):
            idx = base + l
            ids16 = plsc.load_gather(ids_v, [idx])
            vals = plsc.load_gather(w_v, [ids16])
            return acc + jnp.where(ids16 != 0, vals, 0.0)

        acc = lax.fori_loop(0, SEQ, body_l, jnp.zeros((16,), jnp.float32))
        out_v[pl.ds(g * 16, 16)] = acc + bvec

    pltpu.sync_copy(
        out_v,
        out_hbm.at[pl.ds(c * BATCH + s * ROWS_PER_WORKER, ROWS_PER_WORKER)])


@jax.jit
def _bow_logits(ids_flat, w_flat, b16):
    mesh = plsc.VectorSubcoreMesh(core_axis_name="c", subcore_axis_name="s")
    run = functools.partial(
        pl.kernel,
        mesh=mesh,
        out_type=jax.ShapeDtypeStruct((NUM_CLASSES * BATCH,), jnp.float32),
        scratch_types=[
            pltpu.VMEM((TOKENS_PER_WORKER,), jnp.int32),
            pltpu.VMEM((VOCAB,), jnp.float32),
            pltpu.VMEM((16,), jnp.float32),
            pltpu.VMEM((ROWS_PER_WORKER,), jnp.float32),
        ],
    )(_sc_body)
    return run(ids_flat, w_flat, b16)


def kernel(input_ids, W, b):
    ids_flat = input_ids.astype(jnp.int32).reshape(-1)
    w_flat = W.reshape(-1)
    b16 = jnp.repeat(b, 16)
    out = _bow_logits(ids_flat, w_flat, b16)
    return out.reshape(NUM_CLASSES, BATCH).T


# speedup vs baseline: 23.8624x; 23.8624x over previous
import functools

import jax
import jax.numpy as jnp
from jax import lax
from jax.experimental import pallas as pl
from jax.experimental.pallas import tpu as pltpu
from jax.experimental.pallas import tpu_sc as plsc

BATCH = 1024
SEQ = 200
VOCAB = 100000
NUM_CLASSES = 2
NUM_SUBCORES = 16
ROWS_PER_WORKER = BATCH // NUM_SUBCORES
TOKENS_PER_WORKER = ROWS_PER_WORKER * SEQ
ROW_GROUPS = ROWS_PER_WORKER // 16


def _sc_body(ids_hbm, w_hbm, b_hbm, out_hbm, ids_v, w_v, b_v, out_v):
    c = lax.axis_index("c")
    s = lax.axis_index("s")

    pltpu.sync_copy(w_hbm.at[pl.ds(c * VOCAB, VOCAB)], w_v)
    pltpu.sync_copy(ids_hbm.at[pl.ds(s * TOKENS_PER_WORKER, TOKENS_PER_WORKER)],
                    ids_v)
    pltpu.sync_copy(b_hbm.at[pl.ds(c * 16, 16)], b_v)

    bvec = b_v[...]
    row_off = lax.iota(jnp.int32, 16) * SEQ

    for g in range(ROW_GROUPS):
        base = row_off + g * (16 * SEQ)

        def body_l(l, acc, base=base):
            idx = base + l
            ids16 = plsc.load_gather(ids_v, [idx])
            vals = plsc.load_gather(w_v, [ids16])
            return acc + jnp.where(ids16 != 0, vals, 0.0)

        acc = lax.fori_loop(0, SEQ, body_l, jnp.zeros((16,), jnp.float32))
        out_v[pl.ds(g * 16, 16)] = acc + bvec

    pltpu.sync_copy(
        out_v,
        out_hbm.at[pl.ds(c * BATCH + s * ROWS_PER_WORKER, ROWS_PER_WORKER)])


@jax.jit
def _bow_logits(ids_flat, w_flat, b16):
    mesh = plsc.VectorSubcoreMesh(core_axis_name="c", subcore_axis_name="s")
    run = functools.partial(
        pl.kernel,
        mesh=mesh,
        out_type=jax.ShapeDtypeStruct((NUM_CLASSES * BATCH,), jnp.float32),
        scratch_types=[
            pltpu.VMEM((TOKENS_PER_WORKER,), jnp.int32),
            pltpu.VMEM((VOCAB,), jnp.float32),
            pltpu.VMEM((16,), jnp.float32),
            pltpu.VMEM((ROWS_PER_WORKER,), jnp.float32),
        ],
        compiler_params=pltpu.CompilerParams(needs_layout_passes=False),
    )(_sc_body)
    return run(ids_flat, w_flat, b16)


def kernel(input_ids, W, b):
    ids_flat = input_ids.astype(jnp.int32).reshape(-1)
    w_flat = W.reshape(-1)
    b16 = jnp.repeat(b, 16)
    out = _bow_logits(ids_flat, w_flat, b16)
    return out.reshape(NUM_CLASSES, BATCH).T


# speedup vs baseline: 28.1606x; 1.1801x over previous
import functools

import jax
import jax.numpy as jnp
from jax import lax
from jax.experimental import pallas as pl
from jax.experimental.pallas import tpu as pltpu
from jax.experimental.pallas import tpu_sc as plsc

BATCH = 1024
SEQ = 200
VOCAB = 100000
NUM_CLASSES = 2
NUM_SUBCORES = 16
ROWS_PER_WORKER = BATCH // NUM_SUBCORES
TOKENS_PER_WORKER = ROWS_PER_WORKER * SEQ
ROW_GROUPS = ROWS_PER_WORKER // 16


def _sc_body(ids_hbm, w_hbm, b_hbm, out_hbm, ids_v, w_v, b_v, out_v,
             w_sem, ids_sem):
    c = lax.axis_index("c")
    s = lax.axis_index("s")

    cw = pltpu.async_copy(w_hbm.at[pl.ds(c * VOCAB, VOCAB)], w_v, w_sem)
    ci = pltpu.async_copy(
        ids_hbm.at[pl.ds(s * TOKENS_PER_WORKER, TOKENS_PER_WORKER)],
        ids_v, ids_sem)
    pltpu.sync_copy(b_hbm.at[pl.ds(c * 16, 16)], b_v)
    cw.wait()
    ci.wait()

    lane = lax.iota(jnp.int32, 16)
    w_v[pl.ds(0, 16)] = jnp.where(lane == 0, 0.0, w_v[pl.ds(0, 16)])

    bvec = b_v[...]
    zero = jnp.zeros((16,), jnp.float32)

    def body_l(l, accs):
        base = l * ROWS_PER_WORKER
        out = []
        for g in range(ROW_GROUPS):
            ids16 = ids_v[pl.ds(base + g * 16, 16)]
            out.append(accs[g] + plsc.load_gather(w_v, [ids16]))
        return tuple(out)

    accs = lax.fori_loop(0, SEQ, body_l, (zero,) * ROW_GROUPS)
    for g in range(ROW_GROUPS):
        out_v[pl.ds(g * 16, 16)] = accs[g] + bvec

    pltpu.sync_copy(
        out_v,
        out_hbm.at[pl.ds(c * BATCH + s * ROWS_PER_WORKER, ROWS_PER_WORKER)])


@jax.jit
def _bow_logits(ids_flat, w_flat, b16):
    mesh = plsc.VectorSubcoreMesh(core_axis_name="c", subcore_axis_name="s")
    run = functools.partial(
        pl.kernel,
        mesh=mesh,
        out_type=jax.ShapeDtypeStruct((NUM_CLASSES * BATCH,), jnp.float32),
        scratch_types=[
            pltpu.VMEM((TOKENS_PER_WORKER,), jnp.int32),
            pltpu.VMEM((VOCAB,), jnp.float32),
            pltpu.VMEM((16,), jnp.float32),
            pltpu.VMEM((ROWS_PER_WORKER,), jnp.float32),
            pltpu.SemaphoreType.DMA,
            pltpu.SemaphoreType.DMA,
        ],
        compiler_params=pltpu.CompilerParams(needs_layout_passes=False),
    )(_sc_body)
    return run(ids_flat, w_flat, b16)


def kernel(input_ids, W, b):
    ids_w = input_ids.astype(jnp.int32).reshape(
        NUM_SUBCORES, ROWS_PER_WORKER, SEQ).transpose(0, 2, 1)
    w_flat = W.reshape(-1)
    b16 = jnp.repeat(b, 16)
    out = _bow_logits(ids_w.reshape(-1), w_flat, b16)
    return out.reshape(NUM_CLASSES, BATCH).T
